# baseline (speedup 1.0000x reference)
import functools
import math

import jax
import jax.numpy as jnp
import numpy as np
from jax import lax
from jax.experimental import pallas as pl
from jax.experimental.pallas import tpu as pltpu
from jax.experimental.pallas import tpu_sc as plsc

H, W = 512, 1024
EPS = 1e-05
B, F = 64, 50
N = H * W
FPAD = 56
NSLOT = 25
L = 16


def _row_tables():
    thetas = np.linspace(0.5, H - 0.5, num=H) * math.pi / H
    weight = 1.0 / np.sin(thetas)
    residual = weight % 2
    mask = residual >= 1
    residual[mask] -= 1
    residual[~mask] += 1
    n_ones = (weight - residual).astype(np.int32)
    edge_values = ((weight - n_ones) / 2).astype(np.float32)
    lengths = n_ones + 2
    return lengths.astype(np.int32), edge_values


_LEN_NP, _EV_NP = _row_tables()


HC = 4


def _moments_kernel(a_ref, out_ref):
    ar = a_ref[0, 0].reshape(H // HC // 8, 8, W)
    s1p = jnp.sum(ar, axis=0)
    s2p = jnp.sum(ar * ar, axis=0)
    s1 = jnp.sum(s1p)
    s2 = jnp.sum(s2p)
    h = pl.program_id(1)

    @pl.when(h == 0)
    def _():
        out_ref[0, 0, 0] = s1
        out_ref[0, 0, 1] = s2

    @pl.when(h > 0)
    def _():
        out_ref[0, 0, 0] += s1
        out_ref[0, 0, 1] += s2


def _lane(vec, lane):
    m = lax.broadcasted_iota(jnp.int32, (L,), 0) == lane
    return jnp.max(jnp.where(m, vec, jnp.int32(-2**31 + 1)))


def _sc_kernel(apred, ridx, params, fpars, out, idx_v, par_v, fpar_v, rows_v,
               buf_v, out_v, sem):
    wid = lax.axis_index("s") * 2 + lax.axis_index("c")
    lanes = lax.broadcasted_iota(jnp.int32, (L,), 0)
    zeros_l = jnp.zeros((L,), jnp.float32)

    for z in range(W // L):
        buf_v[NSLOT, pl.ds(z * L, L)] = zeros_l

    def one_sample(b):
        pltpu.sync_copy(ridx.at[b], idx_v)
        pltpu.sync_copy(params.at[b], par_v)
        pltpu.sync_copy(fpars.at[b], fpar_v)
        pltpu.async_copy(apred.at[idx_v], rows_v, sem).wait()

        nsl = par_v[0][3]

        def zrow(r, c):
            def zcol(z, c2):
                buf_v[r, pl.ds(z * L, L)] = zeros_l
                return c2

            return lax.fori_loop(0, W // L, zcol, c, unroll=False)

        lax.fori_loop(0, nsl, zrow, 0, unroll=False)

        def fix_step(f, carry):
            prow = par_v[f]
            left = prow[0]
            kw = prow[1]
            wslot = prow[2]
            frow = fpar_v[f]
            ev = jnp.full((L,), frow[0], jnp.float32)
            chain_vf = jnp.full((L,), frow[1], jnp.float32)
            kw_v = jnp.full((L,), kw, jnp.int32)
            l0 = left & (W - 1)
            l0_v = jnp.full((L,), l0, jnp.int32)

            def chunk_step(c, ccarry):
                cdot, cfm = ccarry
                pos = c * L
                pvec = pos + lanes
                j = (pvec - l0_v) & (W - 1)
                maskf = jnp.where(j < kw_v, 1.0, 0.0)
                aw = rows_v[f, pl.ds(pos, L)]
                raw = buf_v[wslot, pl.ds(pos, L)]
                old = raw * chain_vf * maskf
                val = jnp.where(j == 0, ev,
                                jnp.where(j == kw_v - 1, ev, 1.0))
                delta = (val - old) * maskf
                buf_v[wslot, pl.ds(pos, L)] = val * maskf + raw * (1.0 - maskf)
                return cdot + delta * aw, cfm + delta

            len1 = jnp.minimum(kw, W - l0)
            c_lo = l0 // L
            c_hi = (l0 + len1 + L - 1) // L
            carry2 = lax.fori_loop(c_lo, c_hi, chunk_step, carry,
                                   unroll=False)
            c2_hi = (jnp.maximum(l0 + kw - W, 0) + L - 1) // L
            return lax.fori_loop(0, c2_hi, chunk_step, carry2, unroll=False)

        sdot_acc, sfm_acc = lax.fori_loop(0, F, fix_step, (zeros_l, zeros_l),
                                          unroll=False)
        out_v[pl.ds(0, L)] = sdot_acc
        out_v[pl.ds(L, L)] = sfm_acc
        pltpu.sync_copy(out_v, out.at[pl.ds(b * 2 * L, 2 * L)])

    one_sample(wid * 2)
    one_sample(wid * 2 + 1)


def _sc_fixations(apred_flat, ridx, params, fpars):
    mesh = plsc.VectorSubcoreMesh(core_axis_name="c", subcore_axis_name="s")
    run = functools.partial(
        pl.kernel,
        mesh=mesh,
        out_type=jax.ShapeDtypeStruct((B * 2 * L,), jnp.float32),
        scratch_types=[
            pltpu.VMEM((FPAD,), jnp.int32),
            pltpu.VMEM((F, L), jnp.int32),
            pltpu.VMEM((F, L), jnp.float32),
            pltpu.VMEM((FPAD, W), jnp.float32),
            pltpu.VMEM((NSLOT + 1, W), jnp.float32),
            pltpu.VMEM((2 * L,), jnp.float32),
            pltpu.SemaphoreType.DMA,
        ],
    )(_sc_kernel)
    return run(apred_flat, ridx, params, fpars)


def kernel(y_pred, y_gt):
    tabf = jnp.asarray(np.stack([_LEN_NP.astype(np.float32), _EV_NP], axis=1))

    x_idx = jnp.rint(y_gt[:, :, 0] * (W - 1)).astype(jnp.int32)
    y_idx = jnp.rint(y_gt[:, :, 1] * (H - 1)).astype(jnp.int32)
    g = tabf[y_idx]
    kw = g[:, :, 0].astype(jnp.int32)
    ev = g[:, :, 1]
    left = x_idx - kw // 2
    edge = (y_idx == 0) | (y_idx == H - 1)
    kw = jnp.where(edge, W, kw)
    left = jnp.where(edge, 0, left)
    ev = jnp.where(edge, 1.0, ev)

    jj = jnp.arange(F, dtype=jnp.int32)
    same = y_idx[:, :, None] == y_idx[:, None, :]
    before = jj[None, None, :] < jj[None, :, None]
    has_prev = jnp.any(same & before, axis=2)
    has_next = jnp.any(same & (jj[None, None, :] > jj[None, :, None]), axis=2)
    chained = has_prev | has_next
    chain_first = (chained & jnp.logical_not(has_prev)).astype(jnp.float32)
    first_occ = jnp.min(jnp.where(same & before, jj[None, None, :], F), axis=2)
    first_occ = jnp.minimum(first_occ, jj[None, :])
    occ_le = jnp.where(jj[None, None, :] <= first_occ[:, :, None], 1.0, 0.0)
    slot = jnp.sum(chain_first[:, None, :] * occ_le, axis=2).astype(jnp.int32)
    wslot = jnp.where(chained, slot - 1, NSLOT)
    nslots = jnp.sum(chain_first, axis=1).astype(jnp.int32)
    chainf = jnp.where(chained, 1.0, 0.0)

    params = jnp.stack(
        [left, kw, wslot, jnp.broadcast_to(nslots[:, None], (B, F))], axis=2)
    params = jnp.pad(params, ((0, 0), (0, 0), (0, L - 4)))
    fpars = jnp.pad(jnp.stack([ev, chainf], axis=2),
                    ((0, 0), (0, 0), (0, L - 2)))

    ridx = (jnp.pad(y_idx, ((0, 0), (0, FPAD - F)))
            + jnp.arange(B, dtype=jnp.int32)[:, None] * H)

    apred_flat = y_pred.reshape(B * H, W)
    sc_out = _sc_fixations(apred_flat, ridx, params, fpars).reshape(B, 2, L)

    moments = pl.pallas_call(
        _moments_kernel,
        grid=(B, HC),
        in_specs=[pl.BlockSpec((1, 1, H // HC, W), lambda b, h: (b, 0, h, 0))],
        out_specs=pl.BlockSpec((1, 1, 2), lambda b, h: (b, 0, 0),
                               memory_space=pltpu.SMEM),
        out_shape=jax.ShapeDtypeStruct((B, 1, 2), jnp.float32),
    )(y_pred)

    s1 = moments[:, 0, 0]
    s2 = moments[:, 0, 1]
    sdot = jnp.sum(sc_out[:, 0, :], axis=1)
    sfm = jnp.sum(sc_out[:, 1, :], axis=1)

    mean = s1 / N
    var = (s2 - s1 * s1 / N) / (N - 1)
    std = jnp.sqrt(var)
    denom = std + jnp.where(std < EPS, EPS, 0.0)
    return jnp.mean((sdot - mean * sfm) / (denom * F))


# speedup vs baseline: 1.8494x; 1.8494x over previous
import functools
import math

import jax
import jax.numpy as jnp
import numpy as np
from jax import lax
from jax.experimental import pallas as pl
from jax.experimental.pallas import tpu as pltpu
from jax.experimental.pallas import tpu_sc as plsc

H, W = 512, 1024
EPS = 1e-05
B, F = 64, 50
N = H * W
FPAD = 56
NSLOT = 25
L = 16


def _row_tables():
    thetas = np.linspace(0.5, H - 0.5, num=H) * math.pi / H
    weight = 1.0 / np.sin(thetas)
    residual = weight % 2
    mask = residual >= 1
    residual[mask] -= 1
    residual[~mask] += 1
    n_ones = (weight - residual).astype(np.int32)
    edge_values = ((weight - n_ones) / 2).astype(np.float32)
    lengths = n_ones + 2
    return lengths.astype(np.int32), edge_values


_LEN_NP, _EV_NP = _row_tables()


def _moments_kernel(a_ref, out_ref):
    ar = a_ref[0, 0].reshape(H // 8, 8, W)
    s1p = jnp.sum(ar, axis=0)
    s2p = jnp.sum(ar * ar, axis=0)
    out_ref[0, 0, 0] = jnp.sum(s1p)
    out_ref[0, 0, 1] = jnp.sum(s2p)


def _lane(vec, lane):
    m = lax.broadcasted_iota(jnp.int32, (L,), 0) == lane
    return jnp.max(jnp.where(m, vec, jnp.int32(-2**31 + 1)))


def _sc_kernel(apred, ridx, params, fpars, out, idx_v, par_v, fpar_v, rows_v,
               buf_v, out_v, sem):
    wid = lax.axis_index("s") * 2 + lax.axis_index("c")
    lanes = lax.broadcasted_iota(jnp.int32, (L,), 0)
    zeros_l = jnp.zeros((L,), jnp.float32)

    for z in range(W // L):
        buf_v[NSLOT, pl.ds(z * L, L)] = zeros_l

    def one_sample(b):
        pltpu.sync_copy(ridx.at[b], idx_v)
        pltpu.sync_copy(params.at[b], par_v)
        pltpu.sync_copy(fpars.at[b], fpar_v)
        pltpu.async_copy(apred.at[idx_v], rows_v, sem).wait()

        nsl = par_v[0][3]

        def zrow(r, c):
            def zcol(z, c2):
                buf_v[r, pl.ds(z * L, L)] = zeros_l
                return c2

            return lax.fori_loop(0, W // L, zcol, c, unroll=False)

        lax.fori_loop(0, nsl, zrow, 0, unroll=False)

        def fix_step(f, carry):
            prow = par_v[f]
            left = prow[0]
            kw = prow[1]
            wslot = prow[2]
            frow = fpar_v[f]
            ev = jnp.full((L,), frow[0], jnp.float32)
            chain_vf = jnp.full((L,), frow[1], jnp.float32)
            kw_v = jnp.full((L,), kw, jnp.int32)
            l0 = left & (W - 1)
            l0_v = jnp.full((L,), l0, jnp.int32)

            def chunk_step(c, ccarry):
                cdot, cfm = ccarry
                pos = c * L
                pvec = pos + lanes
                j = (pvec - l0_v) & (W - 1)
                maskf = jnp.where(j < kw_v, 1.0, 0.0)
                aw = rows_v[f, pl.ds(pos, L)]
                raw = buf_v[wslot, pl.ds(pos, L)]
                old = raw * chain_vf * maskf
                val = jnp.where(j == 0, ev,
                                jnp.where(j == kw_v - 1, ev, 1.0))
                delta = (val - old) * maskf
                buf_v[wslot, pl.ds(pos, L)] = val * maskf + raw * (1.0 - maskf)
                return cdot + delta * aw, cfm + delta

            len1 = jnp.minimum(kw, W - l0)
            c_lo = l0 // L
            c_hi = (l0 + len1 + L - 1) // L
            carry2 = lax.fori_loop(c_lo, c_hi, chunk_step, carry,
                                   unroll=False)
            c2_hi = (jnp.maximum(l0 + kw - W, 0) + L - 1) // L
            return lax.fori_loop(0, c2_hi, chunk_step, carry2, unroll=False)

        sdot_acc, sfm_acc = lax.fori_loop(0, F, fix_step, (zeros_l, zeros_l),
                                          unroll=False)
        out_v[pl.ds(0, L)] = sdot_acc
        out_v[pl.ds(L, L)] = sfm_acc
        pltpu.sync_copy(out_v, out.at[pl.ds(b * 2 * L, 2 * L)])

    one_sample(wid * 2)
    one_sample(wid * 2 + 1)


def _sc_fixations(apred_flat, ridx, params, fpars):
    mesh = plsc.VectorSubcoreMesh(core_axis_name="c", subcore_axis_name="s")
    run = functools.partial(
        pl.kernel,
        mesh=mesh,
        out_type=jax.ShapeDtypeStruct((B * 2 * L,), jnp.float32),
        scratch_types=[
            pltpu.VMEM((FPAD,), jnp.int32),
            pltpu.VMEM((F, L), jnp.int32),
            pltpu.VMEM((F, L), jnp.float32),
            pltpu.VMEM((FPAD, W), jnp.float32),
            pltpu.VMEM((NSLOT + 1, W), jnp.float32),
            pltpu.VMEM((2 * L,), jnp.float32),
            pltpu.SemaphoreType.DMA,
        ],
    )(_sc_kernel)
    return run(apred_flat, ridx, params, fpars)


def kernel(y_pred, y_gt):
    tabf = jnp.asarray(np.stack([_LEN_NP.astype(np.float32), _EV_NP], axis=1))

    x_idx = jnp.rint(y_gt[:, :, 0] * (W - 1)).astype(jnp.int32)
    y_idx = jnp.rint(y_gt[:, :, 1] * (H - 1)).astype(jnp.int32)
    g = tabf[y_idx]
    kw = g[:, :, 0].astype(jnp.int32)
    ev = g[:, :, 1]
    left = x_idx - kw // 2
    edge = (y_idx == 0) | (y_idx == H - 1)
    kw = jnp.where(edge, W, kw)
    left = jnp.where(edge, 0, left)
    ev = jnp.where(edge, 1.0, ev)

    jj = jnp.arange(F, dtype=jnp.int32)
    same = y_idx[:, :, None] == y_idx[:, None, :]
    before = jj[None, None, :] < jj[None, :, None]
    has_prev = jnp.any(same & before, axis=2)
    has_next = jnp.any(same & (jj[None, None, :] > jj[None, :, None]), axis=2)
    chained = has_prev | has_next
    chain_first = (chained & jnp.logical_not(has_prev)).astype(jnp.float32)
    first_occ = jnp.min(jnp.where(same & before, jj[None, None, :], F), axis=2)
    first_occ = jnp.minimum(first_occ, jj[None, :])
    occ_le = jnp.where(jj[None, None, :] <= first_occ[:, :, None], 1.0, 0.0)
    slot = jnp.sum(chain_first[:, None, :] * occ_le, axis=2).astype(jnp.int32)
    wslot = jnp.where(chained, slot - 1, NSLOT)
    nslots = jnp.sum(chain_first, axis=1).astype(jnp.int32)
    chainf = jnp.where(chained, 1.0, 0.0)

    params = jnp.stack(
        [left, kw, wslot, jnp.broadcast_to(nslots[:, None], (B, F))], axis=2)
    params = jnp.pad(params, ((0, 0), (0, 0), (0, L - 4)))
    fpars = jnp.pad(jnp.stack([ev, chainf], axis=2),
                    ((0, 0), (0, 0), (0, L - 2)))

    ridx = (jnp.pad(y_idx, ((0, 0), (0, FPAD - F)))
            + jnp.arange(B, dtype=jnp.int32)[:, None] * H)

    apred_flat = y_pred.reshape(B * H, W)
    sc_out = _sc_fixations(apred_flat, ridx, params, fpars).reshape(B, 2, L)

    moments = pl.pallas_call(
        _moments_kernel,
        grid=(B,),
        in_specs=[pl.BlockSpec((1, 1, H, W), lambda b: (b, 0, 0, 0))],
        out_specs=pl.BlockSpec((1, 1, 2), lambda b: (b, 0, 0),
                               memory_space=pltpu.SMEM),
        out_shape=jax.ShapeDtypeStruct((B, 1, 2), jnp.float32),
    )(y_pred)

    s1 = moments[:, 0, 0]
    s2 = moments[:, 0, 1]
    sdot = jnp.sum(sc_out[:, 0, :], axis=1)
    sfm = jnp.sum(sc_out[:, 1, :], axis=1)

    mean = s1 / N
    var = (s2 - s1 * s1 / N) / (N - 1)
    std = jnp.sqrt(var)
    denom = std + jnp.where(std < EPS, EPS, 0.0)
    return jnp.mean((sdot - mean * sfm) / (denom * F))


# speedup vs baseline: 1.8557x; 1.0034x over previous
import functools
import math

import jax
import jax.numpy as jnp
import numpy as np
from jax import lax
from jax.experimental import pallas as pl
from jax.experimental.pallas import tpu as pltpu
from jax.experimental.pallas import tpu_sc as plsc

H, W = 512, 1024
EPS = 1e-05
B, F = 64, 50
N = H * W
FPAD = 56
NSLOT = 25
L = 16


def _row_tables():
    thetas = np.linspace(0.5, H - 0.5, num=H) * math.pi / H
    weight = 1.0 / np.sin(thetas)
    residual = weight % 2
    mask = residual >= 1
    residual[mask] -= 1
    residual[~mask] += 1
    n_ones = (weight - residual).astype(np.int32)
    edge_values = ((weight - n_ones) / 2).astype(np.float32)
    lengths = n_ones + 2
    return lengths.astype(np.int32), edge_values


_LEN_NP, _EV_NP = _row_tables()


def _moments_kernel(a_ref, b_ref, out_ref):
    ar = a_ref[0, 0].reshape(H // 16, 8, W)
    br = b_ref[0, 0].reshape(H // 16, 8, W)
    s1p = jnp.sum(ar, axis=0) + jnp.sum(br, axis=0)
    s2p = jnp.sum(ar * ar, axis=0) + jnp.sum(br * br, axis=0)
    out_ref[0, 0, 0] = jnp.sum(s1p)
    out_ref[0, 0, 1] = jnp.sum(s2p)


def _lane(vec, lane):
    m = lax.broadcasted_iota(jnp.int32, (L,), 0) == lane
    return jnp.max(jnp.where(m, vec, jnp.int32(-2**31 + 1)))


def _sc_kernel(apred, ridx, params, fpars, out, idx_v, par_v, fpar_v, rows_v,
               buf_v, out_v, sem):
    wid = lax.axis_index("s") * 2 + lax.axis_index("c")
    lanes = lax.broadcasted_iota(jnp.int32, (L,), 0)
    zeros_l = jnp.zeros((L,), jnp.float32)

    for z in range(W // L):
        buf_v[NSLOT, pl.ds(z * L, L)] = zeros_l

    def one_sample(b):
        pltpu.sync_copy(ridx.at[b], idx_v)
        pltpu.sync_copy(params.at[b], par_v)
        pltpu.sync_copy(fpars.at[b], fpar_v)
        pltpu.async_copy(apred.at[idx_v], rows_v, sem).wait()

        nsl = par_v[0][3]

        def zrow(r, c):
            def zcol(z, c2):
                buf_v[r, pl.ds(z * L, L)] = zeros_l
                return c2

            return lax.fori_loop(0, W // L, zcol, c, unroll=False)

        lax.fori_loop(0, nsl, zrow, 0, unroll=False)

        def fix_step(f, carry):
            prow = par_v[f]
            left = prow[0]
            kw = prow[1]
            wslot = prow[2]
            frow = fpar_v[f]
            ev = jnp.full((L,), frow[0], jnp.float32)
            chain_vf = jnp.full((L,), frow[1], jnp.float32)
            kw_v = jnp.full((L,), kw, jnp.int32)
            l0 = left & (W - 1)
            l0_v = jnp.full((L,), l0, jnp.int32)

            def chunk_step(c, ccarry):
                cdot, cfm = ccarry
                pos = c * L
                pvec = pos + lanes
                j = (pvec - l0_v) & (W - 1)
                maskf = jnp.where(j < kw_v, 1.0, 0.0)
                aw = rows_v[f, pl.ds(pos, L)]
                raw = buf_v[wslot, pl.ds(pos, L)]
                old = raw * chain_vf * maskf
                val = jnp.where(j == 0, ev,
                                jnp.where(j == kw_v - 1, ev, 1.0))
                delta = (val - old) * maskf
                buf_v[wslot, pl.ds(pos, L)] = val * maskf + raw * (1.0 - maskf)
                return cdot + delta * aw, cfm + delta

            len1 = jnp.minimum(kw, W - l0)
            c_lo = l0 // L
            c_hi = (l0 + len1 + L - 1) // L
            carry2 = lax.fori_loop(c_lo, c_hi, chunk_step, carry,
                                   unroll=False)
            c2_hi = (jnp.maximum(l0 + kw - W, 0) + L - 1) // L
            return lax.fori_loop(0, c2_hi, chunk_step, carry2, unroll=False)

        sdot_acc, sfm_acc = lax.fori_loop(0, F, fix_step, (zeros_l, zeros_l),
                                          unroll=False)
        out_v[pl.ds(0, L)] = sdot_acc
        out_v[pl.ds(L, L)] = sfm_acc
        pltpu.sync_copy(out_v, out.at[pl.ds(b * 2 * L, 2 * L)])

    one_sample(wid * 2)
    one_sample(wid * 2 + 1)


def _sc_fixations(apred_flat, ridx, params, fpars):
    mesh = plsc.VectorSubcoreMesh(core_axis_name="c", subcore_axis_name="s")
    run = functools.partial(
        pl.kernel,
        mesh=mesh,
        out_type=jax.ShapeDtypeStruct((B * 2 * L,), jnp.float32),
        scratch_types=[
            pltpu.VMEM((FPAD,), jnp.int32),
            pltpu.VMEM((F, L), jnp.int32),
            pltpu.VMEM((F, L), jnp.float32),
            pltpu.VMEM((FPAD, W), jnp.float32),
            pltpu.VMEM((NSLOT + 1, W), jnp.float32),
            pltpu.VMEM((2 * L,), jnp.float32),
            pltpu.SemaphoreType.DMA,
        ],
    )(_sc_kernel)
    return run(apred_flat, ridx, params, fpars)


def kernel(y_pred, y_gt):
    tabf = jnp.asarray(np.stack([_LEN_NP.astype(np.float32), _EV_NP], axis=1))

    x_idx = jnp.rint(y_gt[:, :, 0] * (W - 1)).astype(jnp.int32)
    y_idx = jnp.rint(y_gt[:, :, 1] * (H - 1)).astype(jnp.int32)
    g = tabf[y_idx]
    kw = g[:, :, 0].astype(jnp.int32)
    ev = g[:, :, 1]
    left = x_idx - kw // 2
    edge = (y_idx == 0) | (y_idx == H - 1)
    kw = jnp.where(edge, W, kw)
    left = jnp.where(edge, 0, left)
    ev = jnp.where(edge, 1.0, ev)

    jj = jnp.arange(F, dtype=jnp.int32)
    same = y_idx[:, :, None] == y_idx[:, None, :]
    before = jj[None, None, :] < jj[None, :, None]
    has_prev = jnp.any(same & before, axis=2)
    has_next = jnp.any(same & (jj[None, None, :] > jj[None, :, None]), axis=2)
    chained = has_prev | has_next
    chain_first = (chained & jnp.logical_not(has_prev)).astype(jnp.float32)
    first_occ = jnp.min(jnp.where(same & before, jj[None, None, :], F), axis=2)
    first_occ = jnp.minimum(first_occ, jj[None, :])
    occ_le = jnp.where(jj[None, None, :] <= first_occ[:, :, None], 1.0, 0.0)
    slot = jnp.sum(chain_first[:, None, :] * occ_le, axis=2).astype(jnp.int32)
    wslot = jnp.where(chained, slot - 1, NSLOT)
    nslots = jnp.sum(chain_first, axis=1).astype(jnp.int32)
    chainf = jnp.where(chained, 1.0, 0.0)

    params = jnp.stack(
        [left, kw, wslot, jnp.broadcast_to(nslots[:, None], (B, F))], axis=2)
    params = jnp.pad(params, ((0, 0), (0, 0), (0, L - 4)))
    fpars = jnp.pad(jnp.stack([ev, chainf], axis=2),
                    ((0, 0), (0, 0), (0, L - 2)))

    ridx = (jnp.pad(y_idx, ((0, 0), (0, FPAD - F)))
            + jnp.arange(B, dtype=jnp.int32)[:, None] * H)

    apred_flat = y_pred.reshape(B * H, W)
    sc_out = _sc_fixations(apred_flat, ridx, params, fpars).reshape(B, 2, L)

    moments = pl.pallas_call(
        _moments_kernel,
        grid=(B,),
        in_specs=[
            pl.BlockSpec((1, 1, H // 2, W), lambda b: (b, 0, 0, 0)),
            pl.BlockSpec((1, 1, H // 2, W), lambda b: (b, 0, 1, 0)),
        ],
        out_specs=pl.BlockSpec((1, 1, 2), lambda b: (b, 0, 0),
                               memory_space=pltpu.SMEM),
        out_shape=jax.ShapeDtypeStruct((B, 1, 2), jnp.float32),
    )(y_pred, y_pred)

    s1 = moments[:, 0, 0]
    s2 = moments[:, 0, 1]
    sdot = jnp.sum(sc_out[:, 0, :], axis=1)
    sfm = jnp.sum(sc_out[:, 1, :], axis=1)

    mean = s1 / N
    var = (s2 - s1 * s1 / N) / (N - 1)
    std = jnp.sqrt(var)
    denom = std + jnp.where(std < EPS, EPS, 0.0)
    return jnp.mean((sdot - mean * sfm) / (denom * F))


# speedup vs baseline: 1.8584x; 1.0014x over previous
import functools
import math

import jax
import jax.numpy as jnp
import numpy as np
from jax import lax
from jax.experimental import pallas as pl
from jax.experimental.pallas import tpu as pltpu
from jax.experimental.pallas import tpu_sc as plsc

H, W = 512, 1024
EPS = 1e-05
B, F = 64, 50
N = H * W
FPAD = 56
NSLOT = 25
L = 16


def _row_tables():
    thetas = np.linspace(0.5, H - 0.5, num=H) * math.pi / H
    weight = 1.0 / np.sin(thetas)
    residual = weight % 2
    mask = residual >= 1
    residual[mask] -= 1
    residual[~mask] += 1
    n_ones = (weight - residual).astype(np.int32)
    edge_values = ((weight - n_ones) / 2).astype(np.float32)
    lengths = n_ones + 2
    return lengths.astype(np.int32), edge_values


_LEN_NP, _EV_NP = _row_tables()


def _moments_kernel(a_ref, b_ref, out_ref):
    ar = a_ref[0, 0].reshape(H // 16, 8, W)
    br = b_ref[0, 0].reshape(H // 16, 8, W)
    s1p = jnp.sum(ar, axis=0) + jnp.sum(br, axis=0)
    s2p = jnp.sum(ar * ar, axis=0) + jnp.sum(br * br, axis=0)
    out_ref[0, 0, 0] = jnp.sum(s1p)
    out_ref[0, 0, 1] = jnp.sum(s2p)


def _sc_kernel(apred, ridx, params, fpars, out, idx_v, par_v, fpar_v, rows_v,
               buf_v, out_v, sem):
    wid = lax.axis_index("s") * 2 + lax.axis_index("c")
    lanes = lax.broadcasted_iota(jnp.int32, (L,), 0)
    zeros_l = jnp.zeros((L,), jnp.float32)

    for z in range(W // L):
        buf_v[NSLOT, pl.ds(z * L, L)] = zeros_l

    def one_sample(b):
        pltpu.sync_copy(ridx.at[b], idx_v)
        pltpu.sync_copy(params.at[b], par_v)
        pltpu.sync_copy(fpars.at[b], fpar_v)
        pltpu.async_copy(apred.at[idx_v], rows_v, sem).wait()

        nsl = par_v[0][3]

        def zrow(r, c):
            def zcol(z, c2):
                buf_v[r, pl.ds(z * L, L)] = zeros_l
                return c2

            return lax.fori_loop(0, W // L, zcol, c, unroll=False)

        lax.fori_loop(0, nsl, zrow, 0, unroll=False)

        def fix_step(f, carry):
            prow = par_v[f]
            left = prow[0]
            kw = prow[1]
            wslot = prow[2]
            frow = fpar_v[f]
            ev = jnp.full((L,), frow[0], jnp.float32)
            chain_vf = jnp.full((L,), frow[1], jnp.float32)
            kw_v = jnp.full((L,), kw, jnp.int32)
            l0 = left & (W - 1)
            l0_v = jnp.full((L,), l0, jnp.int32)

            def chunk_step(c, ccarry):
                cdot, cfm = ccarry
                pos = c * L
                pvec = pos + lanes
                j = (pvec - l0_v) & (W - 1)
                maskf = jnp.where(j < kw_v, 1.0, 0.0)
                aw = rows_v[f, pl.ds(pos, L)]
                raw = buf_v[wslot, pl.ds(pos, L)]
                old = raw * chain_vf * maskf
                val = jnp.where(j == 0, ev,
                                jnp.where(j == kw_v - 1, ev, 1.0))
                delta = (val - old) * maskf
                buf_v[wslot, pl.ds(pos, L)] = val * maskf + raw * (1.0 - maskf)
                return cdot + delta * aw, cfm + delta

            len1 = jnp.minimum(kw, W - l0)
            c_lo = l0 // L
            c_hi = (l0 + len1 + L - 1) // L
            carry2 = lax.fori_loop(c_lo, c_hi, chunk_step, carry,
                                   unroll=False)
            c2_hi = (jnp.maximum(l0 + kw - W, 0) + L - 1) // L
            return lax.fori_loop(0, c2_hi, chunk_step, carry2, unroll=False)

        sdot_acc, sfm_acc = lax.fori_loop(0, F, fix_step, (zeros_l, zeros_l),
                                          unroll=False)
        out_v[pl.ds(0, L)] = sdot_acc
        out_v[pl.ds(L, L)] = sfm_acc
        pltpu.sync_copy(out_v, out.at[pl.ds(b * 2 * L, 2 * L)])

    one_sample(wid * 2)
    one_sample(wid * 2 + 1)


def _sc_fixations(apred_flat, ridx, params, fpars):
    mesh = plsc.VectorSubcoreMesh(core_axis_name="c", subcore_axis_name="s")
    run = functools.partial(
        pl.kernel,
        mesh=mesh,
        out_type=jax.ShapeDtypeStruct((B * 2 * L,), jnp.float32),
        scratch_types=[
            pltpu.VMEM((FPAD,), jnp.int32),
            pltpu.VMEM((F, L), jnp.int32),
            pltpu.VMEM((F, L), jnp.float32),
            pltpu.VMEM((FPAD, W), jnp.float32),
            pltpu.VMEM((NSLOT + 1, W), jnp.float32),
            pltpu.VMEM((2 * L,), jnp.float32),
            pltpu.SemaphoreType.DMA,
        ],
    )(_sc_kernel)
    return run(apred_flat, ridx, params, fpars)


def kernel(y_pred, y_gt):
    tabf = jnp.asarray(np.stack([_LEN_NP.astype(np.float32), _EV_NP], axis=1))

    x_idx = jnp.rint(y_gt[:, :, 0] * (W - 1)).astype(jnp.int32)
    y_idx = jnp.rint(y_gt[:, :, 1] * (H - 1)).astype(jnp.int32)
    g = tabf[y_idx]
    kw = g[:, :, 0].astype(jnp.int32)
    ev = g[:, :, 1]
    left = x_idx - kw // 2
    edge = (y_idx == 0) | (y_idx == H - 1)
    kw = jnp.where(edge, W, kw)
    left = jnp.where(edge, 0, left)
    ev = jnp.where(edge, 1.0, ev)

    jj = jnp.arange(F, dtype=jnp.int32)
    same = y_idx[:, :, None] == y_idx[:, None, :]
    before = jj[None, None, :] < jj[None, :, None]
    has_prev = jnp.any(same & before, axis=2)
    has_next = jnp.any(same & (jj[None, None, :] > jj[None, :, None]), axis=2)
    chained = has_prev | has_next
    chain_first = (chained & jnp.logical_not(has_prev)).astype(jnp.float32)
    first_occ = jnp.min(jnp.where(same & before, jj[None, None, :], F), axis=2)
    first_occ = jnp.minimum(first_occ, jj[None, :])
    occ_le = jnp.where(jj[None, None, :] <= first_occ[:, :, None], 1.0, 0.0)
    slot = jnp.sum(chain_first[:, None, :] * occ_le, axis=2).astype(jnp.int32)
    wslot = jnp.where(chained, slot - 1, NSLOT)
    nslots = jnp.sum(chain_first, axis=1).astype(jnp.int32)
    chainf = jnp.where(chained, 1.0, 0.0)

    params = jnp.stack(
        [left, kw, wslot, jnp.broadcast_to(nslots[:, None], (B, F))], axis=2)
    params = jnp.pad(params, ((0, 0), (0, 0), (0, L - 4)))
    fpars = jnp.pad(jnp.stack([ev, chainf], axis=2),
                    ((0, 0), (0, 0), (0, L - 2)))

    ridx = (jnp.pad(y_idx, ((0, 0), (0, FPAD - F)))
            + jnp.arange(B, dtype=jnp.int32)[:, None] * H)

    apred_flat = y_pred.reshape(B * H, W)
    sc_out = _sc_fixations(apred_flat, ridx, params, fpars).reshape(B, 2, L)

    moments = pl.pallas_call(
        _moments_kernel,
        grid=(B,),
        in_specs=[
            pl.BlockSpec((1, 1, H // 2, W), lambda b: (b, 0, 0, 0)),
            pl.BlockSpec((1, 1, H // 2, W), lambda b: (b, 0, 1, 0)),
        ],
        out_specs=pl.BlockSpec((1, 1, 2), lambda b: (b, 0, 0),
                               memory_space=pltpu.SMEM),
        out_shape=jax.ShapeDtypeStruct((B, 1, 2), jnp.float32),
    )(y_pred, y_pred)

    s1 = moments[:, 0, 0]
    s2 = moments[:, 0, 1]
    sdot = jnp.sum(sc_out[:, 0, :], axis=1)
    sfm = jnp.sum(sc_out[:, 1, :], axis=1)

    mean = s1 / N
    var = (s2 - s1 * s1 / N) / (N - 1)
    std = jnp.sqrt(var)
    denom = std + jnp.where(std < EPS, EPS, 0.0)
    return jnp.mean((sdot - mean * sfm) / (denom * F))
